# baseline (speedup 1.0000x reference)
import jax
import jax.numpy as jnp
from jax.experimental import pallas as pl
from jax.experimental.pallas import tpu as pltpu


def _body(B, mask_ref, hidden_ref, out_ref, sem):
    copies = []
    for b in range(B):
        last = jnp.sum(mask_ref[b, :]) - 1
        copies.append(
            pltpu.make_async_copy(
                hidden_ref.at[b, pl.ds(last, 1), :],
                out_ref.at[pl.ds(b, 1), :],
                sem,
            )
        )
    for c in copies:
        c.start()
    for c in copies:
        c.wait()


def kernel(last_hidden_state, attention_mask):
    B, T, H = last_hidden_state.shape
    mask = attention_mask.astype(jnp.int32)
    return pl.pallas_call(
        lambda *refs: _body(B, *refs),
        out_shape=jax.ShapeDtypeStruct((B, H), jnp.float32),
        in_specs=[
            pl.BlockSpec(memory_space=pltpu.VMEM),
            pl.BlockSpec(memory_space=pl.ANY),
        ],
        out_specs=pl.BlockSpec(memory_space=pl.ANY),
        scratch_shapes=[pltpu.SemaphoreType.DMA],
    )(mask, last_hidden_state)


# speedup vs baseline: 1.6136x; 1.6136x over previous
import jax
import jax.numpy as jnp
from jax.experimental import pallas as pl
from jax.experimental.pallas import tpu as pltpu


def _body(B, mask_ref, hidden_ref, out_ref, sem):
    copies = []
    for b in range(B):
        last = jnp.sum(mask_ref[b, :]) - 1
        copies.append(
            pltpu.make_async_copy(
                hidden_ref.at[b, pl.ds(last, 1), :],
                out_ref.at[pl.ds(b, 1), :],
                sem,
            )
        )
    for c in copies:
        c.start()
    for c in copies:
        c.wait()


def kernel(last_hidden_state, attention_mask):
    B, T, H = last_hidden_state.shape
    mask = attention_mask.astype(jnp.int32)
    return pl.pallas_call(
        lambda *refs: _body(B, *refs),
        out_shape=jax.ShapeDtypeStruct((B, H), jnp.float32),
        in_specs=[
            pl.BlockSpec(memory_space=pltpu.VMEM),
            pl.BlockSpec(memory_space=pl.ANY),
        ],
        out_specs=pl.BlockSpec(memory_space=pltpu.VMEM),
        scratch_shapes=[pltpu.SemaphoreType.DMA],
    )(mask, last_hidden_state)
